# Initial kernel scaffold; baseline (speedup 1.0000x reference)
import math
import functools

import numpy as np
import jax
import jax.numpy as jnp
from jax.experimental import pallas as pl
from jax.experimental.pallas import tpu as pltpu

L = 2048
DM = 1024
H = 16
DK = 64
TOPK = 15
HBLK = 4
CBLK = HBLK * DK


def _build_dft_mats():
    t = np.arange(L, dtype=np.float64)
    f_all = np.arange(L // 2 + 1, dtype=np.float64)
    ang = 2.0 * np.pi * np.outer(f_all, t) / L
    cos = np.cos(ang)
    sin = np.sin(ang[1:-1])
    fwd = np.concatenate([cos, sin], axis=0)

    inv = np.empty((L, L), dtype=np.float64)
    inv[:, 0] = 1.0 / L
    inv[:, 1:1024] = (2.0 / L) * cos[1:-1].T
    inv[:, 1024] = cos[-1] / L
    inv[:, 1025:] = -(2.0 / L) * sin.T
    return fwd.astype(np.float32), inv.astype(np.float32)


_FWD_NP, _INV_NP = _build_dft_mats()
_HSUM_NP = (np.arange(CBLK)[:, None] // DK ==
            np.arange(HBLK)[None, :]).astype(np.float32)


def _mm(a, b):
    return jax.lax.dot(a, b, precision=jax.lax.Precision.HIGHEST,
                       preferred_element_type=jnp.float32)


def _matmul_body(x_ref, w_ref, b_ref, o_ref):
    o_ref[...] = _mm(x_ref[...], w_ref[...]) + b_ref[...]


def _proj(x2d, w, brow, n_blk):
    m, kdim = x2d.shape
    n = w.shape[1]
    grid = (m // 1024, n // n_blk)
    return pl.pallas_call(
        _matmul_body,
        grid=grid,
        in_specs=[
            pl.BlockSpec((1024, kdim), lambda i, j: (i, 0)),
            pl.BlockSpec((kdim, n_blk), lambda i, j: (0, j)),
            pl.BlockSpec((1, n_blk), lambda i, j: (0, j)),
        ],
        out_specs=pl.BlockSpec((1024, n_blk), lambda i, j: (i, j)),
        out_shape=jax.ShapeDtypeStruct((m, n), jnp.float32),
    )(x2d, w, brow)


def _corr_body(q_ref, k_ref, fwd_ref, inv_ref, hsum_ref, o_ref):
    qf = _mm(fwd_ref[...], q_ref[...])
    kf = _mm(fwd_ref[...], k_ref[...])
    aq, bq = qf[:1024, :], qf[1024:, :]
    ak, bk = kf[:1024, :], kf[1024:, :]
    hs = hsum_ref[...]
    p = _mm(aq * ak, hs)
    r = _mm(bq * bk, hs)
    im = _mm(aq * bk - bq * ak, hs)
    row0 = jax.lax.broadcasted_iota(jnp.int32, (1024, HBLK), 0) == 0
    top = jnp.where(row0, p, p + r)
    bot = jnp.where(row0, r, im)
    res = jnp.concatenate([top, bot], axis=0)
    o_ref[0] = _mm(inv_ref[...], res) * (1.0 / DK)


def _corr(qkv):
    grid = (2, DM // CBLK)
    return pl.pallas_call(
        _corr_body,
        grid=grid,
        in_specs=[
            pl.BlockSpec((L, CBLK), lambda b, j: (b, j)),
            pl.BlockSpec((L, CBLK), lambda b, j: (b, j + DM // CBLK)),
            pl.BlockSpec((L, L), lambda b, j: (0, 0)),
            pl.BlockSpec((L, L), lambda b, j: (0, 0)),
            pl.BlockSpec((CBLK, HBLK), lambda b, j: (0, 0)),
        ],
        out_specs=pl.BlockSpec((1, L, HBLK), lambda b, j: (b, 0, j)),
        out_shape=jax.ShapeDtypeStruct((2, L, H), jnp.float32),
    )(qkv, qkv, jnp.asarray(_FWD_NP), jnp.asarray(_INV_NP),
      jnp.asarray(_HSUM_NP))


def _topk_body(c_ref, d_ref, w_ref):
    c = c_ref[0]
    iot = jax.lax.broadcasted_iota(jnp.int32, (L, H), 0)
    vals = []
    for i in range(TOPK):
        m = jnp.max(c, axis=0)
        am = jnp.min(jnp.where(c == m[None, :], iot, L), axis=0)
        vals.append(m)
        d_ref[0, i, :] = am
        c = jnp.where(iot == am[None, :], -jnp.inf, c)
    d_ref[0, TOPK, :] = jnp.zeros((H,), jnp.int32)
    v = jnp.stack(vals, axis=0)
    e = jnp.exp(v - v[0:1, :])
    w = e / jnp.sum(e, axis=0, keepdims=True)
    w_ref[0, :TOPK, :] = w
    w_ref[0, TOPK, :] = jnp.zeros((H,), jnp.float32)


def _topk(corr):
    return pl.pallas_call(
        _topk_body,
        grid=(2,),
        in_specs=[pl.BlockSpec((1, L, H), lambda b: (b, 0, 0))],
        out_specs=[
            pl.BlockSpec((1, TOPK + 1, H), lambda b: (b, 0, 0)),
            pl.BlockSpec((1, TOPK + 1, H), lambda b: (b, 0, 0)),
        ],
        out_shape=[
            jax.ShapeDtypeStruct((2, TOPK + 1, H), jnp.int32),
            jax.ShapeDtypeStruct((2, TOPK + 1, H), jnp.float32),
        ],
    )(corr)


def _agg_body(d_ref, w_ref, v_ref, o_ref, vv_ref):
    b = pl.program_id(0)
    h = pl.program_id(1)
    v = v_ref[...]
    vv_ref[:L, :] = v
    vv_ref[L:, :] = v
    acc = jnp.zeros((L, DK), jnp.float32)
    for kk in range(TOPK):
        d = d_ref[b, kk, h]
        w = w_ref[b, kk, h]
        acc = acc + w * vv_ref[pl.ds(L - d, L), :]
    o_ref[...] = acc


def _agg(qkv, delays, weights):
    grid_spec = pltpu.PrefetchScalarGridSpec(
        num_scalar_prefetch=2,
        grid=(2, H),
        in_specs=[pl.BlockSpec((L, DK), lambda b, h, dr, wr: (b, 32 + h))],
        out_specs=pl.BlockSpec((L, DK), lambda b, h, dr, wr: (b, h)),
        scratch_shapes=[pltpu.VMEM((2 * L, DK), jnp.float32)],
    )
    return pl.pallas_call(
        _agg_body,
        grid_spec=grid_spec,
        out_shape=jax.ShapeDtypeStruct((2 * L, DM), jnp.float32),
    )(delays, weights, qkv)


@jax.jit
def kernel(x, W_q, b_q, W_k, b_k, W_v, b_v, W_o, b_o):
    B, Lx, dm = x.shape
    x2d = x.reshape(B * Lx, dm)
    wqkv = jnp.concatenate([W_q.T, W_k.T, W_v.T], axis=1)
    bqkv = jnp.concatenate([b_q, b_k, b_v]).reshape(1, 3 * dm)
    qkv = _proj(x2d, wqkv, bqkv, 512)
    corr = _corr(qkv)
    delays, weights = _topk(corr)
    context = _agg(qkv, delays, weights)
    out = _proj(context, W_o.T, b_o.reshape(1, dm), 512)
    return out.reshape(B, Lx, dm)


# speedup vs baseline: 10.4200x; 10.4200x over previous
import math
import functools

import numpy as np
import jax
import jax.numpy as jnp
from jax.experimental import pallas as pl
from jax.experimental.pallas import tpu as pltpu

L = 2048
DM = 1024
H = 16
DK = 64
TOPK = 15
HBLK = 4
CBLK = HBLK * DK


def _build_dft_mats():
    t = np.arange(L, dtype=np.float64)
    f_all = np.arange(L // 2 + 1, dtype=np.float64)
    ang = 2.0 * np.pi * np.outer(f_all, t) / L
    cos = np.cos(ang)
    sin = np.sin(ang[1:-1])
    fwd = np.concatenate([cos, sin], axis=0)

    inv = np.empty((L, L), dtype=np.float64)
    inv[:, 0] = 1.0 / L
    inv[:, 1:1024] = (2.0 / L) * cos[1:-1].T
    inv[:, 1024] = cos[-1] / L
    inv[:, 1025:] = -(2.0 / L) * sin.T
    return fwd.astype(np.float32), inv.astype(np.float32)


_FWD_NP, _INV_NP = _build_dft_mats()
_HSUM_NP = (np.arange(CBLK)[:, None] // DK ==
            np.arange(HBLK)[None, :]).astype(np.float32)


def _mm(a, b):
    return jax.lax.dot(a, b, precision=jax.lax.Precision.HIGHEST,
                       preferred_element_type=jnp.float32)


def _mm_default(a, b):
    return jax.lax.dot(a, b, precision=jax.lax.Precision.DEFAULT,
                       preferred_element_type=jnp.float32)


def _matmul_body(x_ref, w_ref, b_ref, o_ref):
    o_ref[...] = _mm_default(x_ref[...], w_ref[...]) + b_ref[...]


def _proj(x2d, w, brow, n_blk):
    m, kdim = x2d.shape
    n = w.shape[1]
    grid = (m // 1024, n // n_blk)
    return pl.pallas_call(
        _matmul_body,
        grid=grid,
        in_specs=[
            pl.BlockSpec((1024, kdim), lambda i, j: (i, 0)),
            pl.BlockSpec((kdim, n_blk), lambda i, j: (0, j)),
            pl.BlockSpec((1, n_blk), lambda i, j: (0, j)),
        ],
        out_specs=pl.BlockSpec((1024, n_blk), lambda i, j: (i, j)),
        out_shape=jax.ShapeDtypeStruct((m, n), jnp.float32),
    )(x2d, w, brow)


FBLK = 1024


def _dft_body(x_ref, fwd_ref, o_ref):
    o_ref[0] = _mm(fwd_ref[...], x_ref[...])


def _dft(qkv):
    grid = (2, L // FBLK, 2 * DM // CBLK)
    return pl.pallas_call(
        _dft_body,
        grid=grid,
        in_specs=[
            pl.BlockSpec((L, CBLK), lambda b, i, c: (b, c)),
            pl.BlockSpec((FBLK, L), lambda b, i, c: (i, 0)),
        ],
        out_specs=pl.BlockSpec((1, FBLK, CBLK), lambda b, i, c: (b, i, c)),
        out_shape=jax.ShapeDtypeStruct((2, L, 2 * DM), jnp.float32),
    )(qkv, jnp.asarray(_FWD_NP))


NJ = DM // CBLK


def _xspec_body(q_ref, k_ref, hsum_ref, o_ref):
    qf = q_ref[0]
    kf = k_ref[0]
    aq, bq = qf[:1024, :], qf[1024:, :]
    ak, bk = kf[:1024, :], kf[1024:, :]
    hs = hsum_ref[...]
    p = _mm(aq * ak, hs)
    r = _mm(bq * bk, hs)
    im = _mm(aq * bk - bq * ak, hs)
    row0 = jax.lax.broadcasted_iota(jnp.int32, (1024, HBLK), 0) == 0
    top = jnp.where(row0, p, p + r)
    bot = jnp.where(row0, r, im)
    o_ref[0, 0] = jnp.concatenate([top, bot], axis=0)


def _xspec(qkf):
    return pl.pallas_call(
        _xspec_body,
        grid=(2, NJ),
        in_specs=[
            pl.BlockSpec((1, L, CBLK), lambda b, j: (b, 0, j)),
            pl.BlockSpec((1, L, CBLK), lambda b, j: (b, 0, j + NJ)),
            pl.BlockSpec((CBLK, HBLK), lambda b, j: (0, 0)),
        ],
        out_specs=pl.BlockSpec((1, 1, L, HBLK), lambda b, j: (b, j, 0, 0)),
        out_shape=jax.ShapeDtypeStruct((2, NJ, L, HBLK), jnp.float32),
    )(qkf, qkf, jnp.asarray(_HSUM_NP))


def _inv_body(r_ref, inv_ref, o_ref):
    o_ref[0, 0] = _mm(inv_ref[...], r_ref[0, 0]) * (1.0 / DK)


def _corr(res):
    return pl.pallas_call(
        _inv_body,
        grid=(2, NJ),
        in_specs=[
            pl.BlockSpec((1, 1, L, HBLK), lambda b, j: (b, j, 0, 0)),
            pl.BlockSpec((L, L), lambda b, j: (0, 0)),
        ],
        out_specs=pl.BlockSpec((1, 1, L, HBLK), lambda b, j: (b, j, 0, 0)),
        out_shape=jax.ShapeDtypeStruct((2, NJ, L, HBLK), jnp.float32),
    )(res, jnp.asarray(_INV_NP))


def _topk_body(c_ref, d_ref, w_ref):
    c = c_ref[0, 0]
    iot = jax.lax.broadcasted_iota(jnp.int32, (L, HBLK), 0)
    vals = []
    for i in range(TOPK):
        m = jnp.max(c, axis=0)
        am = jnp.min(jnp.where(c == m[None, :], iot, L), axis=0)
        vals.append(m)
        d_ref[0, 0, i, :] = am
        c = jnp.where(iot == am[None, :], -jnp.inf, c)
    d_ref[0, 0, TOPK, :] = jnp.zeros((HBLK,), jnp.int32)
    v = jnp.stack(vals, axis=0)
    e = jnp.exp(v - v[0:1, :])
    w = e / jnp.sum(e, axis=0, keepdims=True)
    w_ref[0, 0, :TOPK, :] = w
    w_ref[0, 0, TOPK, :] = jnp.zeros((HBLK,), jnp.float32)


def _topk(corr):
    return pl.pallas_call(
        _topk_body,
        grid=(2, NJ),
        in_specs=[pl.BlockSpec((1, 1, L, HBLK), lambda b, j: (b, j, 0, 0))],
        out_specs=[
            pl.BlockSpec((1, 1, TOPK + 1, HBLK), lambda b, j: (b, j, 0, 0)),
            pl.BlockSpec((1, 1, TOPK + 1, HBLK), lambda b, j: (b, j, 0, 0)),
        ],
        out_shape=[
            jax.ShapeDtypeStruct((2, NJ, TOPK + 1, HBLK), jnp.int32),
            jax.ShapeDtypeStruct((2, NJ, TOPK + 1, HBLK), jnp.float32),
        ],
    )(corr)


def _agg_body(d_ref, w_ref, v_ref, o_ref, va_ref, vb_ref):
    b = pl.program_id(0)
    hh = pl.program_id(1)
    jj = hh // 2
    p0 = 2 * (hh % 2)
    v = v_ref[...]
    va_ref[:L, :] = v[:, :DK]
    va_ref[L:, :] = v[:, :DK]
    vb_ref[:L, :] = v[:, DK:]
    vb_ref[L:, :] = v[:, DK:]
    acc_a = jnp.zeros((L, DK), jnp.float32)
    acc_b = jnp.zeros((L, DK), jnp.float32)
    for kk in range(TOPK):
        d0 = d_ref[b, jj, kk, p0]
        w0 = w_ref[b, jj, kk, p0]
        d1 = d_ref[b, jj, kk, p0 + 1]
        w1 = w_ref[b, jj, kk, p0 + 1]
        acc_a = acc_a + w0 * va_ref[pl.ds(L - d0, L), :]
        acc_b = acc_b + w1 * vb_ref[pl.ds(L - d1, L), :]
    o_ref[...] = jnp.concatenate([acc_a, acc_b], axis=1)


def _agg(qkv, delays, weights):
    grid_spec = pltpu.PrefetchScalarGridSpec(
        num_scalar_prefetch=2,
        grid=(2, H // 2),
        in_specs=[pl.BlockSpec((L, 2 * DK),
                               lambda b, hh, dr, wr: (b, 16 + hh))],
        out_specs=pl.BlockSpec((L, 2 * DK), lambda b, hh, dr, wr: (b, hh)),
        scratch_shapes=[pltpu.VMEM((2 * L, DK), jnp.float32),
                        pltpu.VMEM((2 * L, DK), jnp.float32)],
    )
    return pl.pallas_call(
        _agg_body,
        grid_spec=grid_spec,
        out_shape=jax.ShapeDtypeStruct((2 * L, DM), jnp.float32),
    )(delays, weights, qkv)


@jax.jit
def kernel(x, W_q, b_q, W_k, b_k, W_v, b_v, W_o, b_o):
    B, Lx, dm = x.shape
    x2d = x.reshape(B * Lx, dm)
    wqkv = jnp.concatenate([W_q.T, W_k.T, W_v.T], axis=1)
    bqkv = jnp.concatenate([b_q, b_k, b_v]).reshape(1, 3 * dm)
    qkv = _proj(x2d, wqkv, bqkv, 512)
    qkf = _dft(qkv)
    corr = _corr(_xspec(qkf))
    delays, weights = _topk(corr)
    context = _agg(qkv, delays, weights)
    out = _proj(context, W_o.T, b_o.reshape(1, dm), 512)
    return out.reshape(B, Lx, dm)


# speedup vs baseline: 11.8082x; 1.1332x over previous
import math
import functools

import numpy as np
import jax
import jax.numpy as jnp
from jax.experimental import pallas as pl
from jax.experimental.pallas import tpu as pltpu

L = 2048
DM = 1024
H = 16
DK = 64
TOPK = 15
HBLK = 4
CBLK = HBLK * DK


def _build_dft_mats():
    t = np.arange(L, dtype=np.float64)
    f_all = np.arange(L // 2 + 1, dtype=np.float64)
    ang = 2.0 * np.pi * np.outer(f_all, t) / L
    cos = np.cos(ang)
    sin = np.sin(ang[1:-1])
    fwd = np.concatenate([cos, sin], axis=0)

    inv = np.empty((L, L), dtype=np.float64)
    inv[:, 0] = 1.0 / L
    inv[:, 1:1024] = (2.0 / L) * cos[1:-1].T
    inv[:, 1024] = cos[-1] / L
    inv[:, 1025:] = -(2.0 / L) * sin.T
    return fwd.astype(np.float32), inv.astype(np.float32)


_FWD_NP, _INV_NP = _build_dft_mats()
_HSUM_NP = (np.arange(CBLK)[:, None] // DK ==
            np.arange(HBLK)[None, :]).astype(np.float32)


def _mm(a, b):
    return jax.lax.dot(a, b, precision=jax.lax.Precision.HIGHEST,
                       preferred_element_type=jnp.float32)


def _mm_default(a, b):
    return jax.lax.dot(a, b, precision=jax.lax.Precision.DEFAULT,
                       preferred_element_type=jnp.float32)


def _matmul_body(x_ref, w_ref, b_ref, o_ref):
    o_ref[...] = _mm_default(x_ref[...], w_ref[...]) + b_ref[...]


def _proj(x2d, w, brow, n_blk):
    m, kdim = x2d.shape
    n = w.shape[1]
    grid = (m // 1024, n // n_blk)
    return pl.pallas_call(
        _matmul_body,
        grid=grid,
        in_specs=[
            pl.BlockSpec((1024, kdim), lambda i, j: (i, 0)),
            pl.BlockSpec((kdim, n_blk), lambda i, j: (0, j)),
            pl.BlockSpec((1, n_blk), lambda i, j: (0, j)),
        ],
        out_specs=pl.BlockSpec((1024, n_blk), lambda i, j: (i, j)),
        out_shape=jax.ShapeDtypeStruct((m, n), jnp.float32),
    )(x2d, w, brow)


FBLK = 1024


def _dft_body(x_ref, fwd_ref, o_ref):
    o_ref[0] = _mm(fwd_ref[...], x_ref[...])


def _dft(qkv):
    grid = (2, L // FBLK, 2 * DM // CBLK)
    return pl.pallas_call(
        _dft_body,
        grid=grid,
        in_specs=[
            pl.BlockSpec((L, CBLK), lambda b, i, c: (b, c)),
            pl.BlockSpec((FBLK, L), lambda b, i, c: (i, 0)),
        ],
        out_specs=pl.BlockSpec((1, FBLK, CBLK), lambda b, i, c: (b, i, c)),
        out_shape=jax.ShapeDtypeStruct((2, L, 2 * DM), jnp.float32),
    )(qkv, jnp.asarray(_FWD_NP))


NJ = DM // CBLK


def _xspec_body(q_ref, k_ref, hsum_ref, o_ref):
    qf = q_ref[0]
    kf = k_ref[0]
    aq, bq = qf[:1024, :], qf[1024:, :]
    ak, bk = kf[:1024, :], kf[1024:, :]
    hs = hsum_ref[...]
    p = _mm(aq * ak, hs)
    r = _mm(bq * bk, hs)
    im = _mm(aq * bk - bq * ak, hs)
    row0 = jax.lax.broadcasted_iota(jnp.int32, (1024, HBLK), 0) == 0
    top = jnp.where(row0, p, p + r)
    bot = jnp.where(row0, r, im)
    o_ref[0, 0] = jnp.concatenate([top, bot], axis=0)


def _xspec(qkf):
    return pl.pallas_call(
        _xspec_body,
        grid=(2, NJ),
        in_specs=[
            pl.BlockSpec((1, L, CBLK), lambda b, j: (b, 0, j)),
            pl.BlockSpec((1, L, CBLK), lambda b, j: (b, 0, j + NJ)),
            pl.BlockSpec((CBLK, HBLK), lambda b, j: (0, 0)),
        ],
        out_specs=pl.BlockSpec((1, 1, L, HBLK), lambda b, j: (b, j, 0, 0)),
        out_shape=jax.ShapeDtypeStruct((2, NJ, L, HBLK), jnp.float32),
    )(qkf, qkf, jnp.asarray(_HSUM_NP))


TBLK = 256


def _inv_body(r_ref, inv_ref, o_ref):
    inv = inv_ref[...]
    blocks = []
    for b in range(2):
        for j in range(NJ):
            cb = _mm(inv, r_ref[b, j]) * (1.0 / DK)
            blocks.append(cb.T)
    o_ref[...] = jnp.concatenate(blocks, axis=0)


def _corr(res):
    return pl.pallas_call(
        _inv_body,
        grid=(L // TBLK,),
        in_specs=[
            pl.BlockSpec((2, NJ, L, HBLK), lambda t: (0, 0, 0, 0)),
            pl.BlockSpec((TBLK, L), lambda t: (t, 0)),
        ],
        out_specs=pl.BlockSpec((2 * H, TBLK), lambda t: (0, t)),
        out_shape=jax.ShapeDtypeStruct((2 * H, L), jnp.float32),
    )(res, jnp.asarray(_INV_NP))


def _topk_body(c_ref, d_ref, w_ref):
    c = c_ref[...]
    rows = 2 * H
    iot = jax.lax.broadcasted_iota(jnp.int32, (rows, L), 1)
    vals, idxs = [], []
    for i in range(TOPK):
        m = jnp.max(c, axis=1, keepdims=True)
        am = jnp.min(jnp.where(c == m, iot, L), axis=1, keepdims=True)
        vals.append(m)
        idxs.append(am)
        c = jnp.where(iot == am, -jnp.inf, c)
    v = jnp.concatenate(vals, axis=1)
    d = jnp.concatenate(idxs, axis=1)
    e = jnp.exp(v - v[:, 0:1])
    w = e / jnp.sum(e, axis=1, keepdims=True)
    zi = jnp.zeros((rows, 1), jnp.int32)
    zf = jnp.zeros((rows, 1), jnp.float32)
    d_ref[...] = jnp.concatenate([d, zi], axis=1)
    w_ref[...] = jnp.concatenate([w, zf], axis=1)


def _topk(corr):
    rows = 2 * H
    return pl.pallas_call(
        _topk_body,
        in_specs=[pl.BlockSpec((rows, L), lambda: (0, 0))],
        out_specs=[
            pl.BlockSpec((rows, TOPK + 1), lambda: (0, 0)),
            pl.BlockSpec((rows, TOPK + 1), lambda: (0, 0)),
        ],
        out_shape=[
            jax.ShapeDtypeStruct((rows, TOPK + 1), jnp.int32),
            jax.ShapeDtypeStruct((rows, TOPK + 1), jnp.float32),
        ],
    )(corr)


def _agg_body(d_ref, w_ref, v_ref, o_ref, va_ref, vb_ref):
    b = pl.program_id(0)
    hh = pl.program_id(1)
    v = v_ref[...]
    va_ref[:L, :] = v[:, :DK]
    va_ref[L:, :] = v[:, :DK]
    vb_ref[:L, :] = v[:, DK:]
    vb_ref[L:, :] = v[:, DK:]
    r0 = b * H + 2 * hh
    acc_a = jnp.zeros((L, DK), jnp.float32)
    acc_b = jnp.zeros((L, DK), jnp.float32)
    for kk in range(TOPK):
        d0 = d_ref[r0, kk]
        w0 = w_ref[r0, kk]
        d1 = d_ref[r0 + 1, kk]
        w1 = w_ref[r0 + 1, kk]
        acc_a = acc_a + w0 * va_ref[pl.ds(L - d0, L), :]
        acc_b = acc_b + w1 * vb_ref[pl.ds(L - d1, L), :]
    o_ref[...] = jnp.concatenate([acc_a, acc_b], axis=1)


def _agg(qkv, delays, weights):
    grid_spec = pltpu.PrefetchScalarGridSpec(
        num_scalar_prefetch=2,
        grid=(2, H // 2),
        in_specs=[pl.BlockSpec((L, 2 * DK),
                               lambda b, hh, dr, wr: (b, 16 + hh))],
        out_specs=pl.BlockSpec((L, 2 * DK), lambda b, hh, dr, wr: (b, hh)),
        scratch_shapes=[pltpu.VMEM((2 * L, DK), jnp.float32),
                        pltpu.VMEM((2 * L, DK), jnp.float32)],
    )
    return pl.pallas_call(
        _agg_body,
        grid_spec=grid_spec,
        out_shape=jax.ShapeDtypeStruct((2 * L, DM), jnp.float32),
    )(delays, weights, qkv)


@jax.jit
def kernel(x, W_q, b_q, W_k, b_k, W_v, b_v, W_o, b_o):
    B, Lx, dm = x.shape
    x2d = x.reshape(B * Lx, dm)
    wqkv = jnp.concatenate([W_q.T, W_k.T, W_v.T], axis=1)
    bqkv = jnp.concatenate([b_q, b_k, b_v]).reshape(1, 3 * dm)
    qkv = _proj(x2d, wqkv, bqkv, 512)
    qkf = _dft(qkv)
    corr = _corr(_xspec(qkf))
    delays, weights = _topk(corr)
    context = _agg(qkv, delays, weights)
    out = _proj(context, W_o.T, b_o.reshape(1, dm), 512)
    return out.reshape(B, Lx, dm)


# speedup vs baseline: 13.9021x; 1.1773x over previous
import math
import functools

import numpy as np
import jax
import jax.numpy as jnp
from jax.experimental import pallas as pl
from jax.experimental.pallas import tpu as pltpu

L = 2048
DM = 1024
H = 16
DK = 64
TOPK = 15
HBLK = 4
CBLK = HBLK * DK


def _build_dft_mats():
    t = np.arange(L, dtype=np.float64)
    f_all = np.arange(L // 2 + 1, dtype=np.float64)
    ang = 2.0 * np.pi * np.outer(f_all, t) / L
    cos = np.cos(ang)
    sin = np.sin(ang[1:-1])
    fwd = np.concatenate([cos, sin], axis=0)

    inv = np.empty((L, L), dtype=np.float64)
    inv[:, 0] = 1.0 / L
    inv[:, 1:1024] = (2.0 / L) * cos[1:-1].T
    inv[:, 1024] = cos[-1] / L
    inv[:, 1025:] = -(2.0 / L) * sin.T
    return fwd.astype(np.float32), inv.astype(np.float32)


_FWD_NP, _INV_NP = _build_dft_mats()
_HSUM_NP = (np.arange(CBLK)[:, None] // DK ==
            np.arange(HBLK)[None, :]).astype(np.float32)


def _mm(a, b):
    return jax.lax.dot(a, b, precision=jax.lax.Precision.HIGHEST,
                       preferred_element_type=jnp.float32)


def _mm_default(a, b):
    return jax.lax.dot(a, b, precision=jax.lax.Precision.DEFAULT,
                       preferred_element_type=jnp.float32)


def _matmul_body(x_ref, w_ref, b_ref, o_ref):
    o_ref[...] = _mm_default(x_ref[...], w_ref[...]) + b_ref[...]


def _proj(x2d, w, brow, n_blk):
    m, kdim = x2d.shape
    n = w.shape[1]
    grid = (m // 1024, n // n_blk)
    return pl.pallas_call(
        _matmul_body,
        grid=grid,
        in_specs=[
            pl.BlockSpec((1024, kdim), lambda i, j: (i, 0)),
            pl.BlockSpec((kdim, n_blk), lambda i, j: (0, j)),
            pl.BlockSpec((1, n_blk), lambda i, j: (0, j)),
        ],
        out_specs=pl.BlockSpec((1024, n_blk), lambda i, j: (i, j)),
        out_shape=jax.ShapeDtypeStruct((m, n), jnp.float32),
    )(x2d, w, brow)


FBLK = 1024


def _dft_body(x_ref, fwd_ref, o_ref):
    o_ref[0] = _mm(fwd_ref[...], x_ref[...])


def _dft(qkv):
    grid = (2, L // FBLK, 2 * DM // CBLK)
    return pl.pallas_call(
        _dft_body,
        grid=grid,
        in_specs=[
            pl.BlockSpec((L, CBLK), lambda b, i, c: (b, c)),
            pl.BlockSpec((FBLK, L), lambda b, i, c: (i, 0)),
        ],
        out_specs=pl.BlockSpec((1, FBLK, CBLK), lambda b, i, c: (b, i, c)),
        out_shape=jax.ShapeDtypeStruct((2, L, 2 * DM), jnp.float32),
    )(qkv, jnp.asarray(_FWD_NP))


NJ = DM // CBLK


def _xspec_body(q_ref, k_ref, hsum_ref, o_ref):
    qf = q_ref[0]
    kf = k_ref[0]
    aq, bq = qf[:1024, :], qf[1024:, :]
    ak, bk = kf[:1024, :], kf[1024:, :]
    hs = hsum_ref[...]
    p = _mm(aq * ak, hs)
    r = _mm(bq * bk, hs)
    im = _mm(aq * bk - bq * ak, hs)
    row0 = jax.lax.broadcasted_iota(jnp.int32, (1024, HBLK), 0) == 0
    top = jnp.where(row0, p, p + r)
    bot = jnp.where(row0, r, im)
    o_ref[0, 0] = jnp.concatenate([top, bot], axis=0)


def _xspec(qkf):
    return pl.pallas_call(
        _xspec_body,
        grid=(2, NJ),
        in_specs=[
            pl.BlockSpec((1, L, CBLK), lambda b, j: (b, 0, j)),
            pl.BlockSpec((1, L, CBLK), lambda b, j: (b, 0, j + NJ)),
            pl.BlockSpec((CBLK, HBLK), lambda b, j: (0, 0)),
        ],
        out_specs=pl.BlockSpec((1, 1, L, HBLK), lambda b, j: (b, j, 0, 0)),
        out_shape=jax.ShapeDtypeStruct((2, NJ, L, HBLK), jnp.float32),
    )(qkf, qkf, jnp.asarray(_HSUM_NP))


TBLK = 256


def _inv_body(r_ref, inv_ref, o_ref):
    inv = inv_ref[...]
    res = jnp.concatenate(
        [r_ref[b, j] for b in range(2) for j in range(NJ)],
        axis=1)
    cb = _mm(inv, res) * (1.0 / DK)
    o_ref[...] = cb.T


def _corr(res):
    return pl.pallas_call(
        _inv_body,
        grid=(L // TBLK,),
        in_specs=[
            pl.BlockSpec((2, NJ, L, HBLK), lambda t: (0, 0, 0, 0)),
            pl.BlockSpec((TBLK, L), lambda t: (t, 0)),
        ],
        out_specs=pl.BlockSpec((2 * H, TBLK), lambda t: (0, t)),
        out_shape=jax.ShapeDtypeStruct((2 * H, L), jnp.float32),
    )(res, jnp.asarray(_INV_NP))


def _topk_body(c_ref, d_ref, w_ref):
    c = c_ref[...]
    rows = 2 * H
    iot = jax.lax.broadcasted_iota(jnp.int32, (rows, L), 1)
    vals, idxs = [], []
    for i in range(TOPK):
        m = jnp.max(c, axis=1, keepdims=True)
        am = jnp.min(jnp.where(c == m, iot, L), axis=1, keepdims=True)
        vals.append(m)
        idxs.append(am)
        c = jnp.where(iot == am, -jnp.inf, c)
    v = jnp.concatenate(vals, axis=1)
    d = jnp.concatenate(idxs, axis=1)
    e = jnp.exp(v - v[:, 0:1])
    w = e / jnp.sum(e, axis=1, keepdims=True)
    zi = jnp.zeros((rows, 1), jnp.int32)
    zf = jnp.zeros((rows, 1), jnp.float32)
    d_ref[...] = jnp.concatenate([d, zi], axis=1)
    w_ref[...] = jnp.concatenate([w, zf], axis=1)


def _topk(corr):
    rows = 2 * H
    return pl.pallas_call(
        _topk_body,
        in_specs=[pl.BlockSpec((rows, L), lambda: (0, 0))],
        out_specs=[
            pl.BlockSpec((rows, TOPK + 1), lambda: (0, 0)),
            pl.BlockSpec((rows, TOPK + 1), lambda: (0, 0)),
        ],
        out_shape=[
            jax.ShapeDtypeStruct((rows, TOPK + 1), jnp.int32),
            jax.ShapeDtypeStruct((rows, TOPK + 1), jnp.float32),
        ],
    )(corr)


def _agg_body(d_ref, w_ref, v_ref, o_ref, va_ref, vb_ref):
    b = pl.program_id(0)
    hh = pl.program_id(1)
    v = v_ref[...]
    va_ref[:L, :] = v[:, :DK]
    va_ref[L:, :] = v[:, :DK]
    vb_ref[:L, :] = v[:, DK:]
    vb_ref[L:, :] = v[:, DK:]
    r0 = b * H + 2 * hh
    acc_a = jnp.zeros((L, DK), jnp.float32)
    acc_b = jnp.zeros((L, DK), jnp.float32)
    for kk in range(TOPK):
        d0 = d_ref[r0, kk]
        w0 = w_ref[r0, kk]
        d1 = d_ref[r0 + 1, kk]
        w1 = w_ref[r0 + 1, kk]
        acc_a = acc_a + w0 * va_ref[pl.ds(L - d0, L), :]
        acc_b = acc_b + w1 * vb_ref[pl.ds(L - d1, L), :]
    o_ref[...] = jnp.concatenate([acc_a, acc_b], axis=1)


def _agg(qkv, delays, weights):
    grid_spec = pltpu.PrefetchScalarGridSpec(
        num_scalar_prefetch=2,
        grid=(2, H // 2),
        in_specs=[pl.BlockSpec((L, 2 * DK),
                               lambda b, hh, dr, wr: (b, 16 + hh))],
        out_specs=pl.BlockSpec((L, 2 * DK), lambda b, hh, dr, wr: (b, hh)),
        scratch_shapes=[pltpu.VMEM((2 * L, DK), jnp.float32),
                        pltpu.VMEM((2 * L, DK), jnp.float32)],
    )
    return pl.pallas_call(
        _agg_body,
        grid_spec=grid_spec,
        out_shape=jax.ShapeDtypeStruct((2 * L, DM), jnp.float32),
    )(delays, weights, qkv)


@jax.jit
def kernel(x, W_q, b_q, W_k, b_k, W_v, b_v, W_o, b_o):
    B, Lx, dm = x.shape
    x2d = x.reshape(B * Lx, dm)
    wqkv = jnp.concatenate([W_q.T, W_k.T, W_v.T], axis=1)
    bqkv = jnp.concatenate([b_q, b_k, b_v]).reshape(1, 3 * dm)
    qkv = _proj(x2d, wqkv, bqkv, 512)
    qkf = _dft(qkv)
    corr = _corr(_xspec(qkf))
    delays, weights = _topk(corr)
    context = _agg(qkv, delays, weights)
    out = _proj(context, W_o.T, b_o.reshape(1, dm), 512)
    return out.reshape(B, Lx, dm)
